# baseline (speedup 1.0000x reference)
import functools

import jax
import jax.numpy as jnp
from jax import lax
from jax.experimental import pallas as pl
from jax.experimental.pallas import tpu as pltpu
from jax.experimental.pallas import tpu_sc as plsc

_CHUNK = 128


def _sc_gather(idx, emb, *, chunks):
    nrows = idx.shape[0]
    embed_dim = emb.shape[1]
    mesh = plsc.VectorSubcoreMesh(core_axis_name="c", subcore_axis_name="s")
    num_cores = mesh.num_cores
    nbuf = 6
    depth = 4

    @functools.partial(
        pl.kernel,
        out_type=jax.ShapeDtypeStruct((nrows, embed_dim), jnp.float32),
        mesh=mesh,
        scratch_types=[
            pltpu.VMEM((chunks * _CHUNK,), jnp.int32),
            pltpu.VMEM((nbuf, _CHUNK, embed_dim), jnp.float32),
            [pltpu.SemaphoreType.DMA] * nbuf,
            [pltpu.SemaphoreType.DMA] * nbuf,
        ],
    )
    def gather_kernel(emb_hbm, idx_hbm, out_hbm, idx_v, rows_v, gsems, ssems):
        wid = lax.axis_index("s") * num_cores + lax.axis_index("c")
        base = wid * (chunks * _CHUNK)
        pltpu.sync_copy(idx_hbm.at[pl.ds(base, chunks * _CHUNK)], idx_v)

        def gather(g):
            return pltpu.async_copy(
                emb_hbm.at[idx_v.at[pl.ds(g * _CHUNK, _CHUNK)]],
                rows_v.at[g % nbuf],
                gsems[g % nbuf],
            )

        def scatter(g):
            return pltpu.async_copy(
                rows_v.at[g % nbuf],
                out_hbm.at[pl.ds(base + g * _CHUNK, _CHUNK)],
                ssems[g % nbuf],
            )

        gath = {g: gather(g) for g in range(min(depth, chunks))}
        scat = {}
        for g in range(chunks):
            gath.pop(g).wait()
            scat[g] = scatter(g)
            nx = g + depth
            if nx < chunks:
                if nx - nbuf in scat:
                    scat.pop(nx - nbuf).wait()
                gath[nx] = gather(nx)
        for d in scat.values():
            d.wait()

    return gather_kernel(emb, idx)


def _fuse_weights(W1, W2, b1, b2):
    def body(w1_ref, w2_ref, b1_ref, b2_ref, wft_ref, bft_ref):
        w2 = w2_ref[...]
        wft_ref[...] = lax.dot_general(
            w2, w1_ref[...],
            dimension_numbers=(((0,), (1,)), ((), ())),
            preferred_element_type=jnp.float32,
            precision=lax.Precision.HIGHEST,
        )
        bft_ref[...] = (
            lax.dot_general(
                w2, b1_ref[...],
                dimension_numbers=(((0,), (0,)), ((), ())),
                preferred_element_type=jnp.float32,
                precision=lax.Precision.HIGHEST,
            )
            + b2_ref[...]
        )

    embed_dim, hidden = W1.shape
    num_out = W2.shape[1]
    return pl.pallas_call(
        body,
        out_shape=(
            jax.ShapeDtypeStruct((num_out, embed_dim), jnp.float32),
            jax.ShapeDtypeStruct((num_out, 1), jnp.float32),
        ),
    )(W1, W2, b1.reshape(hidden, 1), b2.reshape(num_out, 1))


def _mlp_t(gathered, wft, bft, prev, *, block_c, hist, seg, l_off):
    embed_dim = gathered.shape[1]
    batch = gathered.shape[0] // seg
    num_out = wft.shape[0]
    per_l = batch // block_c

    def body(*refs):
        x_ref, wft_ref, bft_ref = refs[0], refs[1], refs[2]
        o_ref = refs[-1]
        xt = x_ref[...].T
        yt = jnp.dot(wft_ref[...], xt, preferred_element_type=jnp.float32)
        o_ref[...] = (yt + bft_ref[...]).reshape(1, num_out, block_c)

    in_specs = [
        pl.BlockSpec((block_c, embed_dim), lambda l, j: (l * per_l + j, 0)),
        pl.BlockSpec((num_out, embed_dim), lambda l, j: (0, 0)),
        pl.BlockSpec((num_out, 1), lambda l, j: (0, 0)),
    ]
    args = [gathered, wft, bft]
    aliases = {}
    if prev is not None:
        in_specs.append(pl.BlockSpec(memory_space=pl.ANY))
        args.append(prev)
        aliases = {3: 0}

    return pl.pallas_call(
        body,
        grid=(seg, per_l),
        in_specs=in_specs,
        out_specs=pl.BlockSpec(
            (1, num_out, block_c), lambda l, j: (l + l_off, 0, j)
        ),
        out_shape=jax.ShapeDtypeStruct((hist, num_out, batch), jnp.float32),
        input_output_aliases=aliases,
    )(*args)


def kernel(input_notes, emb, W1, b1, W2, b2):
    batch, hist = input_notes.shape
    nrows = batch * hist
    info = plsc.get_sparse_core_info()
    nw = info.num_cores * info.num_subcores

    idx = input_notes.T.reshape(nrows).astype(jnp.int32)

    wft, bft = _fuse_weights(W1, W2, b1, b2)

    segments = (6, hist - 6)
    out_t = None
    l_off = 0
    for seg in segments:
        seg_rows = batch * seg
        idx_seg = lax.dynamic_slice_in_dim(idx, l_off * batch, seg_rows)
        gathered = _sc_gather(idx_seg, emb, chunks=seg_rows // (nw * _CHUNK))
        out_t = _mlp_t(
            gathered, wft, bft, out_t,
            block_c=4096, hist=hist, seg=seg, l_off=l_off,
        )
        l_off += seg
    return jnp.transpose(out_t, (2, 0, 1))


# speedup vs baseline: 1.0257x; 1.0257x over previous
import functools

import jax
import jax.numpy as jnp
from jax import lax
from jax.experimental import pallas as pl
from jax.experimental.pallas import tpu as pltpu
from jax.experimental.pallas import tpu_sc as plsc

_CHUNK = 128


def _sc_gather(idx, emb, *, chunks):
    nrows = idx.shape[0]
    embed_dim = emb.shape[1]
    mesh = plsc.VectorSubcoreMesh(core_axis_name="c", subcore_axis_name="s")
    num_cores = mesh.num_cores
    nbuf = 6
    depth = 4

    @functools.partial(
        pl.kernel,
        out_type=jax.ShapeDtypeStruct((nrows, embed_dim), jnp.float32),
        mesh=mesh,
        scratch_types=[
            pltpu.VMEM((chunks * _CHUNK,), jnp.int32),
            pltpu.VMEM((nbuf, _CHUNK, embed_dim), jnp.float32),
            [pltpu.SemaphoreType.DMA] * nbuf,
            [pltpu.SemaphoreType.DMA] * nbuf,
        ],
    )
    def gather_kernel(emb_hbm, idx_hbm, out_hbm, idx_v, rows_v, gsems, ssems):
        wid = lax.axis_index("s") * num_cores + lax.axis_index("c")
        base = wid * (chunks * _CHUNK)
        pltpu.sync_copy(idx_hbm.at[pl.ds(base, chunks * _CHUNK)], idx_v)

        def gather(g):
            return pltpu.async_copy(
                emb_hbm.at[idx_v.at[pl.ds(g * _CHUNK, _CHUNK)]],
                rows_v.at[g % nbuf],
                gsems[g % nbuf],
            )

        def scatter(g):
            return pltpu.async_copy(
                rows_v.at[g % nbuf],
                out_hbm.at[pl.ds(base + g * _CHUNK, _CHUNK)],
                ssems[g % nbuf],
            )

        gath = {g: gather(g) for g in range(min(depth, chunks))}
        scat = {}
        for g in range(chunks):
            gath.pop(g).wait()
            scat[g] = scatter(g)
            nx = g + depth
            if nx < chunks:
                if nx - nbuf in scat:
                    scat.pop(nx - nbuf).wait()
                gath[nx] = gather(nx)
        for d in scat.values():
            d.wait()

    return gather_kernel(emb, idx)


def _fuse_weights(W1, W2, b1, b2):
    def body(w1_ref, w2_ref, b1_ref, b2_ref, wft_ref, bft_ref):
        w2 = w2_ref[...]
        wft_ref[...] = lax.dot_general(
            w2, w1_ref[...],
            dimension_numbers=(((0,), (1,)), ((), ())),
            preferred_element_type=jnp.float32,
            precision=lax.Precision.HIGHEST,
        )
        bft_ref[...] = (
            lax.dot_general(
                w2, b1_ref[...],
                dimension_numbers=(((0,), (0,)), ((), ())),
                preferred_element_type=jnp.float32,
                precision=lax.Precision.HIGHEST,
            )
            + b2_ref[...]
        )

    embed_dim, hidden = W1.shape
    num_out = W2.shape[1]
    return pl.pallas_call(
        body,
        out_shape=(
            jax.ShapeDtypeStruct((num_out, embed_dim), jnp.float32),
            jax.ShapeDtypeStruct((num_out, 1), jnp.float32),
        ),
    )(W1, W2, b1.reshape(hidden, 1), b2.reshape(num_out, 1))


def _mlp_t(gathered, wft, bft, *, block_c, hist):
    nrows, embed_dim = gathered.shape
    batch = nrows // hist
    num_out = wft.shape[0]
    per_l = batch // block_c

    def body(x_ref, wft_ref, bft_ref, o_ref):
        xt = x_ref[...].T
        yt = jnp.dot(wft_ref[...], xt, preferred_element_type=jnp.float32)
        o_ref[...] = (yt + bft_ref[...]).reshape(1, num_out, block_c)

    return pl.pallas_call(
        body,
        grid=(hist, per_l),
        in_specs=[
            pl.BlockSpec((block_c, embed_dim), lambda l, j: (l * per_l + j, 0)),
            pl.BlockSpec((num_out, embed_dim), lambda l, j: (0, 0)),
            pl.BlockSpec((num_out, 1), lambda l, j: (0, 0)),
        ],
        out_specs=pl.BlockSpec((1, num_out, block_c), lambda l, j: (l, 0, j)),
        out_shape=jax.ShapeDtypeStruct((hist, num_out, batch), jnp.float32),
    )(gathered, wft, bft)


def kernel(input_notes, emb, W1, b1, W2, b2):
    batch, hist = input_notes.shape
    nrows = batch * hist
    info = plsc.get_sparse_core_info()
    nw = info.num_cores * info.num_subcores
    chunks = nrows // (nw * _CHUNK)

    idx = input_notes.T.reshape(nrows).astype(jnp.int32)

    gathered = _sc_gather(idx, emb, chunks=chunks)
    wft, bft = _fuse_weights(W1, W2, b1, b2)
    out_t = _mlp_t(gathered, wft, bft, block_c=4096, hist=hist)
    return jnp.transpose(out_t, (2, 0, 1))


# speedup vs baseline: 1.0277x; 1.0019x over previous
import functools

import jax
import jax.numpy as jnp
from jax import lax
from jax.experimental import pallas as pl
from jax.experimental.pallas import tpu as pltpu
from jax.experimental.pallas import tpu_sc as plsc

_CHUNK = 128


def _sc_gather(idx, emb, *, chunks):
    nrows = idx.shape[0]
    embed_dim = emb.shape[1]
    mesh = plsc.VectorSubcoreMesh(core_axis_name="c", subcore_axis_name="s")
    num_cores = mesh.num_cores
    nbuf = 7
    depth = 5

    @functools.partial(
        pl.kernel,
        out_type=jax.ShapeDtypeStruct((nrows, embed_dim), jnp.float32),
        mesh=mesh,
        scratch_types=[
            pltpu.VMEM((chunks * _CHUNK,), jnp.int32),
            pltpu.VMEM((nbuf, _CHUNK, embed_dim), jnp.float32),
            [pltpu.SemaphoreType.DMA] * nbuf,
            [pltpu.SemaphoreType.DMA] * nbuf,
        ],
    )
    def gather_kernel(emb_hbm, idx_hbm, out_hbm, idx_v, rows_v, gsems, ssems):
        wid = lax.axis_index("s") * num_cores + lax.axis_index("c")
        base = wid * (chunks * _CHUNK)
        pltpu.sync_copy(idx_hbm.at[pl.ds(base, chunks * _CHUNK)], idx_v)

        def gather(g):
            return pltpu.async_copy(
                emb_hbm.at[idx_v.at[pl.ds(g * _CHUNK, _CHUNK)]],
                rows_v.at[g % nbuf],
                gsems[g % nbuf],
            )

        def scatter(g):
            return pltpu.async_copy(
                rows_v.at[g % nbuf],
                out_hbm.at[pl.ds(base + g * _CHUNK, _CHUNK)],
                ssems[g % nbuf],
            )

        gath = {g: gather(g) for g in range(min(depth, chunks))}
        scat = {}
        for g in range(chunks):
            gath.pop(g).wait()
            scat[g] = scatter(g)
            nx = g + depth
            if nx < chunks:
                if nx - nbuf in scat:
                    scat.pop(nx - nbuf).wait()
                gath[nx] = gather(nx)
        for d in scat.values():
            d.wait()

    return gather_kernel(emb, idx)


def _fuse_weights(W1, W2, b1, b2):
    def body(w1_ref, w2_ref, b1_ref, b2_ref, wft_ref, bft_ref):
        w2 = w2_ref[...]
        wft_ref[...] = lax.dot_general(
            w2, w1_ref[...],
            dimension_numbers=(((0,), (1,)), ((), ())),
            preferred_element_type=jnp.float32,
            precision=lax.Precision.HIGHEST,
        )
        bft_ref[...] = (
            lax.dot_general(
                w2, b1_ref[...],
                dimension_numbers=(((0,), (0,)), ((), ())),
                preferred_element_type=jnp.float32,
                precision=lax.Precision.HIGHEST,
            )
            + b2_ref[...]
        )

    embed_dim, hidden = W1.shape
    num_out = W2.shape[1]
    return pl.pallas_call(
        body,
        out_shape=(
            jax.ShapeDtypeStruct((num_out, embed_dim), jnp.float32),
            jax.ShapeDtypeStruct((num_out, 1), jnp.float32),
        ),
    )(W1, W2, b1.reshape(hidden, 1), b2.reshape(num_out, 1))


def _mlp_t(gathered, wft, bft, *, block_c, hist):
    nrows, embed_dim = gathered.shape
    batch = nrows // hist
    num_out = wft.shape[0]
    per_l = batch // block_c

    def body(x_ref, wft_ref, bft_ref, o_ref):
        xt = x_ref[...].T
        yt = jnp.dot(wft_ref[...], xt, preferred_element_type=jnp.float32)
        o_ref[...] = (yt + bft_ref[...]).reshape(1, num_out, block_c)

    return pl.pallas_call(
        body,
        grid=(hist, per_l),
        in_specs=[
            pl.BlockSpec((block_c, embed_dim), lambda l, j: (l * per_l + j, 0)),
            pl.BlockSpec((num_out, embed_dim), lambda l, j: (0, 0)),
            pl.BlockSpec((num_out, 1), lambda l, j: (0, 0)),
        ],
        out_specs=pl.BlockSpec((1, num_out, block_c), lambda l, j: (l, 0, j)),
        out_shape=jax.ShapeDtypeStruct((hist, num_out, batch), jnp.float32),
    )(gathered, wft, bft)


def kernel(input_notes, emb, W1, b1, W2, b2):
    batch, hist = input_notes.shape
    nrows = batch * hist
    info = plsc.get_sparse_core_info()
    nw = info.num_cores * info.num_subcores
    chunks = nrows // (nw * _CHUNK)

    idx = input_notes.T.reshape(nrows).astype(jnp.int32)

    gathered = _sc_gather(idx, emb, chunks=chunks)
    wft, bft = _fuse_weights(W1, W2, b1, b2)
    out_t = _mlp_t(gathered, wft, bft, block_c=4096, hist=hist)
    return jnp.transpose(out_t, (2, 0, 1))
